# baseline (speedup 1.0000x reference)
import functools

import jax
import jax.numpy as jnp
from jax import lax
from jax.experimental import pallas as pl
from jax.experimental.pallas import tpu as pltpu
from jax.experimental.pallas import tpu_sc as plsc

_NUM_USERS = 20000
_NUM_ITEMS = 30000
_N = _NUM_USERS + _NUM_ITEMS
_D = 64
_DH = _D // 2
_E = 800000
_CHUNK = 128
_NCHUNKS = _E // _CHUNK
_IB = 10
_NG = _NCHUNKS // _IB
_PAIRS = _IB // 2
_NS = 16
_ROW_CHUNKS = -(-_N // _CHUNK)
_ACC_ROWS = _ROW_CHUNKS * _CHUNK
_FULL_OUT_CHUNKS = _N // _CHUNK
_OUT_REM = _N - _FULL_OUT_CHUNKS * _CHUNK

_GATHER_DNUMS = lax.GatherDimensionNumbers(
    offset_dims=(), collapsed_slice_dims=(0,), start_index_map=(0,))


def _lane_bcast(vec, j):
    idx = jnp.full((16, 1), j, dtype=jnp.int32)
    return lax.gather(vec, idx, _GATHER_DNUMS, (1,),
                      mode=lax.GatherScatterMode.PROMISE_IN_BOUNDS)


def _scale_rows(rows, wvb, ch):
    for g in range(_CHUNK // 16):
        wv = wvb[ch, pl.ds(g * 16, 16)]
        for j in range(16):
            r = g * 16 + j
            wb = _lane_bcast(wv, j)
            for dd in range(_DH // 16):
                sl = pl.ds(dd * 16, 16)
                rows[r, sl] = rows[r, sl] * wb


def _edge_phase(x_hbm, src2d, dst2d, w2d, srcb, dstb, wvb,
                rows0, rows1, acc, gsem, ssem, s):
    ng = (_NG - s + _NS - 1) // _NS

    def _group(i, carry):
        c0 = (s + _NS * i) * _IB
        pltpu.sync_copy(src2d.at[pl.ds(c0, _IB)], srcb)
        pltpu.sync_copy(dst2d.at[pl.ds(c0, _IB)], dstb)
        pltpu.sync_copy(w2d.at[pl.ds(c0, _IB)], wvb)
        pltpu.async_copy(x_hbm.at[srcb.at[0]], rows0, gsem)

        def _pair(p, carry2):
            a = 2 * p
            b = a + 1
            pltpu.make_async_copy(x_hbm.at[srcb.at[a]], rows0, gsem).wait()

            @pl.when(p > 0)
            def _():
                pltpu.make_async_copy(rows1, acc.at[dstb.at[b]], ssem).wait()

            pltpu.async_copy(x_hbm.at[srcb.at[b]], rows1, gsem)
            _scale_rows(rows0, wvb, a)
            pltpu.async_copy(rows0, acc.at[dstb.at[a]], ssem, add=True)
            pltpu.make_async_copy(x_hbm.at[srcb.at[b]], rows1, gsem).wait()
            pltpu.make_async_copy(rows0, acc.at[dstb.at[a]], ssem).wait()

            @pl.when(p < _PAIRS - 1)
            def _():
                pltpu.async_copy(x_hbm.at[srcb.at[a + 2]], rows0, gsem)

            _scale_rows(rows1, wvb, b)
            pltpu.async_copy(rows1, acc.at[dstb.at[b]], ssem, add=True)
            return carry2
        lax.fori_loop(0, _PAIRS, _pair, 0)
        pltpu.make_async_copy(rows1, acc.at[dstb.at[0]], ssem).wait()
        return carry
    lax.fori_loop(0, ng, _group, 0)


def _spmm_body(xlo_hbm, xhi_hbm, src2d, dst2d, w2d,
               outlo_hbm, outhi_hbm,
               srcb, dstb, wvb, rows0, rows1, acc, gsem, ssem):
    c = lax.axis_index("c")
    s = lax.axis_index("s")

    def _zero_rows(r, carry):
        for dd in range(_DH // 16):
            rows0[r, pl.ds(dd * 16, 16)] = jnp.zeros((16,), jnp.float32)
        return carry
    lax.fori_loop(0, _CHUNK, _zero_rows, 0)

    nz = (_ROW_CHUNKS - s + _NS - 1) // _NS
    def _zero_acc(i, carry):
        ct = s + _NS * i
        pltpu.sync_copy(rows0, acc.at[pl.ds(ct * _CHUNK, _CHUNK)])
        return carry
    lax.fori_loop(0, nz, _zero_acc, 0)

    plsc.subcore_barrier()

    @pl.when(c == 0)
    def _():
        _edge_phase(xlo_hbm, src2d, dst2d, w2d, srcb, dstb, wvb,
                    rows0, rows1, acc, gsem, ssem, s)

    @pl.when(c == 1)
    def _():
        _edge_phase(xhi_hbm, src2d, dst2d, w2d, srcb, dstb, wvb,
                    rows0, rows1, acc, gsem, ssem, s)

    plsc.subcore_barrier()

    no = (_FULL_OUT_CHUNKS - s + _NS - 1) // _NS
    def _out_chunk(i, carry):
        r0 = (s + _NS * i) * _CHUNK
        sl = pl.ds(r0, _CHUNK)

        @pl.when(c == 0)
        def _():
            pltpu.sync_copy(acc.at[sl], outlo_hbm.at[sl])

        @pl.when(c == 1)
        def _():
            pltpu.sync_copy(acc.at[sl], outhi_hbm.at[sl])
        return carry
    lax.fori_loop(0, no, _out_chunk, 0)

    @pl.when(s == _FULL_OUT_CHUNKS % _NS)
    def _():
        sl = pl.ds(_FULL_OUT_CHUNKS * _CHUNK, _OUT_REM)

        @pl.when(c == 0)
        def _():
            pltpu.sync_copy(acc.at[sl], outlo_hbm.at[sl])

        @pl.when(c == 1)
        def _():
            pltpu.sync_copy(acc.at[sl], outhi_hbm.at[sl])


_spmm = functools.partial(
    pl.kernel,
    out_type=(
        jax.ShapeDtypeStruct((_N, _DH), jnp.float32),
        jax.ShapeDtypeStruct((_N, _DH), jnp.float32),
    ),
    mesh=plsc.VectorSubcoreMesh(core_axis_name="c", subcore_axis_name="s"),
    compiler_params=pltpu.CompilerParams(use_tc_tiling_on_sc=False),
    scratch_types=[
        pltpu.VMEM((_IB, _CHUNK), jnp.int32),
        pltpu.VMEM((_IB, _CHUNK), jnp.int32),
        pltpu.VMEM((_IB, _CHUNK), jnp.float32),
        pltpu.VMEM((_CHUNK, _DH), jnp.float32),
        pltpu.VMEM((_CHUNK, _DH), jnp.float32),
        pltpu.VMEM_SHARED((_ACC_ROWS, _DH), jnp.float32),
        pltpu.SemaphoreType.DMA,
        pltpu.SemaphoreType.DMA,
    ],
)(_spmm_body)


def _combine_body(x0_ref, lo2, hi2, lo3, hi3, lo4, hi4, o_ref):
    lo = lo2[...] + lo3[...] + lo4[...]
    hi = hi2[...] + hi3[...] + hi4[...]
    o_ref[...] = 0.25 * (x0_ref[...] + jnp.concatenate([lo, hi], axis=1))


_combine = pl.pallas_call(
    _combine_body,
    out_shape=jax.ShapeDtypeStruct((_N, _D), jnp.float32),
    grid=(50,),
    in_specs=[pl.BlockSpec((1000, _D), lambda i: (i, 0))]
    + [pl.BlockSpec((1000, _DH), lambda i: (i, 0))] * 6,
    out_specs=pl.BlockSpec((1000, _D), lambda i: (i, 0)),
)


def kernel(user_emb, item_emb, edge_index, edge_weight):
    x0 = jnp.concatenate([user_emb, item_emb], axis=0)
    src2d = edge_index[0].reshape(_NCHUNKS, _CHUNK)
    dst2d = edge_index[1].reshape(_NCHUNKS, _CHUNK)
    w2d = edge_weight.reshape(_NCHUNKS, _CHUNK)
    lo0, hi0 = x0[:, :_DH], x0[:, _DH:]
    lo1, hi1 = _spmm(lo0, hi0, src2d, dst2d, w2d)
    lo2, hi2 = _spmm(lo1, hi1, src2d, dst2d, w2d)
    lo3, hi3 = _spmm(lo2, hi2, src2d, dst2d, w2d)
    lo4, hi4 = _spmm(lo3, hi3, src2d, dst2d, w2d)
    light = _combine(x0, lo2, hi2, lo3, hi3, lo4, hi4)
    return light[:_NUM_USERS], light[_NUM_USERS:]


# speedup vs baseline: 1.0067x; 1.0067x over previous
import functools

import jax
import jax.numpy as jnp
from jax import lax
from jax.experimental import pallas as pl
from jax.experimental.pallas import tpu as pltpu
from jax.experimental.pallas import tpu_sc as plsc

_NUM_USERS = 20000
_NUM_ITEMS = 30000
_N = _NUM_USERS + _NUM_ITEMS
_D = 64
_DH = _D // 2
_E = 800000
_CHUNK = 128
_NCHUNKS = _E // _CHUNK
_IB = 10
_NG = _NCHUNKS // _IB
_PAIRS = _IB // 2
_NS = 16
_ROW_CHUNKS = -(-_N // _CHUNK)
_ACC_ROWS = _ROW_CHUNKS * _CHUNK
_FULL_OUT_CHUNKS = _N // _CHUNK
_OUT_REM = _N - _FULL_OUT_CHUNKS * _CHUNK

_GATHER_DNUMS = lax.GatherDimensionNumbers(
    offset_dims=(), collapsed_slice_dims=(0,), start_index_map=(0,))


def _lane_bcast(vec, j):
    idx = jnp.full((16, 1), j, dtype=jnp.int32)
    return lax.gather(vec, idx, _GATHER_DNUMS, (1,),
                      mode=lax.GatherScatterMode.PROMISE_IN_BOUNDS)


def _scale_rows(rows, wvb, ch):
    for g in range(_CHUNK // 16):
        wv = wvb[ch, pl.ds(g * 16, 16)]
        for j in range(16):
            r = g * 16 + j
            wb = _lane_bcast(wv, j)
            for dd in range(_DH // 16):
                sl = pl.ds(dd * 16, 16)
                rows[r, sl] = rows[r, sl] * wb


def _spmm_body(x_hbm, src2d, dst2d, w2d, out_hbm,
               srcb, dstb, wvb, rows0, rows1, acc, gsem, ssem):
    c = lax.axis_index("c")
    s = lax.axis_index("s")
    xbase = c * _N

    def _zero_rows(r, carry):
        for dd in range(_DH // 16):
            rows0[r, pl.ds(dd * 16, 16)] = jnp.zeros((16,), jnp.float32)
        return carry
    lax.fori_loop(0, _CHUNK, _zero_rows, 0)

    nz = (_ROW_CHUNKS - s + _NS - 1) // _NS
    def _zero_acc(i, carry):
        ct = s + _NS * i
        pltpu.async_copy(rows0, acc.at[pl.ds(ct * _CHUNK, _CHUNK)], gsem)
        return carry
    lax.fori_loop(0, nz, _zero_acc, 0)

    def _zero_drain(i, carry):
        pltpu.make_async_copy(rows0, acc.at[pl.ds(0, _CHUNK)], gsem).wait()
        return carry
    lax.fori_loop(0, nz, _zero_drain, 0)

    plsc.subcore_barrier()

    ng = (_NG - s + _NS - 1) // _NS

    def _group(i, carry):
        c0 = (s + _NS * i) * _IB
        pltpu.sync_copy(src2d.at[pl.ds(c0, _IB)], srcb)
        pltpu.sync_copy(dst2d.at[pl.ds(c0, _IB)], dstb)
        pltpu.sync_copy(w2d.at[pl.ds(c0, _IB)], wvb)
        for ch in range(_IB):
            for g in range(_CHUNK // 16):
                sl = pl.ds(g * 16, 16)
                srcb[ch, sl] = srcb[ch, sl] + xbase
        pltpu.async_copy(x_hbm.at[srcb.at[0]], rows0, gsem)

        def _pair(p, carry2):
            a = 2 * p
            b = a + 1
            pltpu.make_async_copy(x_hbm.at[srcb.at[a]], rows0, gsem).wait()

            @pl.when(p > 0)
            def _():
                pltpu.make_async_copy(rows1, acc.at[dstb.at[b]], ssem).wait()

            pltpu.async_copy(x_hbm.at[srcb.at[b]], rows1, gsem)
            _scale_rows(rows0, wvb, a)
            pltpu.async_copy(rows0, acc.at[dstb.at[a]], ssem, add=True)
            pltpu.make_async_copy(x_hbm.at[srcb.at[b]], rows1, gsem).wait()
            pltpu.make_async_copy(rows0, acc.at[dstb.at[a]], ssem).wait()

            @pl.when(p < _PAIRS - 1)
            def _():
                pltpu.async_copy(x_hbm.at[srcb.at[a + 2]], rows0, gsem)

            _scale_rows(rows1, wvb, b)
            pltpu.async_copy(rows1, acc.at[dstb.at[b]], ssem, add=True)
            return carry2
        lax.fori_loop(0, _PAIRS, _pair, 0)
        pltpu.make_async_copy(rows1, acc.at[dstb.at[0]], ssem).wait()
        return carry
    lax.fori_loop(0, ng, _group, 0)

    plsc.subcore_barrier()

    no = (_FULL_OUT_CHUNKS - s + _NS - 1) // _NS
    def _out_chunk(i, carry):
        r0 = (s + _NS * i) * _CHUNK
        pltpu.async_copy(acc.at[pl.ds(r0, _CHUNK)],
                         out_hbm.at[pl.ds(xbase + r0, _CHUNK)], gsem)
        return carry
    lax.fori_loop(0, no, _out_chunk, 0)

    @pl.when(s == _FULL_OUT_CHUNKS % _NS)
    def _():
        r0 = _FULL_OUT_CHUNKS * _CHUNK
        pltpu.async_copy(acc.at[pl.ds(r0, _OUT_REM)],
                         out_hbm.at[pl.ds(xbase + r0, _OUT_REM)], gsem)

    def _out_drain(i, carry):
        pltpu.make_async_copy(acc.at[pl.ds(0, _CHUNK)],
                              out_hbm.at[pl.ds(0, _CHUNK)], gsem).wait()
        return carry
    lax.fori_loop(0, no, _out_drain, 0)

    @pl.when(s == _FULL_OUT_CHUNKS % _NS)
    def _():
        pltpu.make_async_copy(acc.at[pl.ds(0, _OUT_REM)],
                              out_hbm.at[pl.ds(0, _OUT_REM)], gsem).wait()


_spmm = functools.partial(
    pl.kernel,
    out_type=jax.ShapeDtypeStruct((2 * _N, _DH), jnp.float32),
    mesh=plsc.VectorSubcoreMesh(core_axis_name="c", subcore_axis_name="s"),
    compiler_params=pltpu.CompilerParams(use_tc_tiling_on_sc=False),
    scratch_types=[
        pltpu.VMEM((_IB, _CHUNK), jnp.int32),
        pltpu.VMEM((_IB, _CHUNK), jnp.int32),
        pltpu.VMEM((_IB, _CHUNK), jnp.float32),
        pltpu.VMEM((_CHUNK, _DH), jnp.float32),
        pltpu.VMEM((_CHUNK, _DH), jnp.float32),
        pltpu.VMEM_SHARED((_ACC_ROWS, _DH), jnp.float32),
        pltpu.SemaphoreType.DMA,
        pltpu.SemaphoreType.DMA,
    ],
)(_spmm_body)


def _combine_body(x0_ref, lo2, hi2, lo3, hi3, lo4, hi4, o_ref):
    lo = lo2[...] + lo3[...] + lo4[...]
    hi = hi2[...] + hi3[...] + hi4[...]
    o_ref[...] = 0.25 * (x0_ref[...] + jnp.concatenate([lo, hi], axis=1))


_LO_SPEC = pl.BlockSpec((1000, _DH), lambda i: (i, 0))
_HI_SPEC = pl.BlockSpec((1000, _DH), lambda i: (i + _N // 1000, 0))

_combine = pl.pallas_call(
    _combine_body,
    out_shape=jax.ShapeDtypeStruct((_N, _D), jnp.float32),
    grid=(50,),
    in_specs=[pl.BlockSpec((1000, _D), lambda i: (i, 0)),
              _LO_SPEC, _HI_SPEC, _LO_SPEC, _HI_SPEC, _LO_SPEC, _HI_SPEC],
    out_specs=pl.BlockSpec((1000, _D), lambda i: (i, 0)),
)


def kernel(user_emb, item_emb, edge_index, edge_weight):
    x0 = jnp.concatenate([user_emb, item_emb], axis=0)
    src2d = edge_index[0].reshape(_NCHUNKS, _CHUNK)
    dst2d = edge_index[1].reshape(_NCHUNKS, _CHUNK)
    w2d = edge_weight.reshape(_NCHUNKS, _CHUNK)
    x0st = jnp.concatenate([x0[:, :_DH], x0[:, _DH:]], axis=0)
    x1st = _spmm(x0st, src2d, dst2d, w2d)
    x2st = _spmm(x1st, src2d, dst2d, w2d)
    x3st = _spmm(x2st, src2d, dst2d, w2d)
    x4st = _spmm(x3st, src2d, dst2d, w2d)
    light = _combine(x0, x2st, x2st, x3st, x3st, x4st, x4st)
    return light[:_NUM_USERS], light[_NUM_USERS:]


# speedup vs baseline: 1.8138x; 1.8017x over previous
import functools

import jax
import jax.numpy as jnp
from jax import lax
from jax.experimental import pallas as pl
from jax.experimental.pallas import tpu as pltpu
from jax.experimental.pallas import tpu_sc as plsc

_NUM_USERS = 20000
_NUM_ITEMS = 30000
_N = _NUM_USERS + _NUM_ITEMS
_D = 64
_DH = _D // 2
_E = 800000
_CHUNK = 128
_NCHUNKS = _E // _CHUNK
_GB = 5
_PAIR_CHUNKS = 2 * _GB
_NPAIRS = _NCHUNKS // _PAIR_CHUNKS
_NS = 16
_ROW_CHUNKS = -(-_N // _CHUNK)
_ACC_ROWS = _ROW_CHUNKS * _CHUNK
_FULL_OUT_CHUNKS = _N // _CHUNK
_OUT_REM = _N - _FULL_OUT_CHUNKS * _CHUNK

_GATHER_DNUMS = lax.GatherDimensionNumbers(
    offset_dims=(), collapsed_slice_dims=(0,), start_index_map=(0,))


def _lane_bcast(vec, j):
    idx = jnp.full((16, 1), j, dtype=jnp.int32)
    return lax.gather(vec, idx, _GATHER_DNUMS, (1,),
                      mode=lax.GatherScatterMode.PROMISE_IN_BOUNDS)


def _scale_chunk(rows, swb, ch):
    def _g(gi, carry):
        wv = lax.bitcast_convert_type(
            swb[ch, pl.ds(128 + gi * 16, 16)], jnp.float32)

        def _j(j, carry2):
            for u in range(4):
                e = gi * 16 + j * 4 + u
                wb = _lane_bcast(wv, j * 4 + u)
                for dd in range(_DH // 16):
                    sl = pl.ds(dd * 16, 16)
                    rows[e, sl] = rows[e, sl] * wb
            return carry2
        lax.fori_loop(0, 4, _j, 0)
        return carry
    lax.fori_loop(0, _CHUNK // 16, _g, 0)


def _spmm_body(x_hbm, sw2d, dst2d, out_hbm,
               swbA, dstbA, swbB, dstbB, r0, r1, r2, r3, r4,
               acc, gsem, ssem, isem):
    c = lax.axis_index("c")
    s = lax.axis_index("s")
    xbase = c * _N
    rbuf = [r0, r1, r2, r3, r4]

    def _shift_src(swb):
        for ch in range(_GB):
            for g in range(_CHUNK // 16):
                sl = pl.ds(g * 16, 16)
                swb[ch, sl] = swb[ch, sl] + xbase

    def _zero_rows(rr, carry):
        for dd in range(_DH // 16):
            r0[rr, pl.ds(dd * 16, 16)] = jnp.zeros((16,), jnp.float32)
        return carry
    lax.fori_loop(0, _CHUNK, _zero_rows, 0)

    nz = (_ROW_CHUNKS - s + _NS - 1) // _NS
    def _zero_acc(i, carry):
        ct = s + _NS * i
        pltpu.async_copy(r0, acc.at[pl.ds(ct * _CHUNK, _CHUNK)], gsem)
        return carry
    lax.fori_loop(0, nz, _zero_acc, 0)

    def _zero_drain(i, carry):
        pltpu.make_async_copy(r0, acc.at[pl.ds(0, _CHUNK)], gsem).wait()
        return carry
    lax.fori_loop(0, nz, _zero_drain, 0)

    plsc.subcore_barrier()

    ni = (_NPAIRS - s + _NS - 1) // _NS

    def _gather_ref(swb, row):
        return x_hbm.at[swb.at[row, pl.ds(0, _CHUNK)]]

    def _issue_gather(x, swb, row):
        pltpu.async_copy(_gather_ref(swb, row), rbuf[x % _GB], gsem)

    def _wait_gather(x, swb, row):
        pltpu.make_async_copy(_gather_ref(swb, row), rbuf[x % _GB], gsem).wait()

    def _issue_scatter(x, dstb, row):
        pltpu.async_copy(rbuf[x % _GB], acc.at[dstb.at[row]], ssem, add=True)

    def _wait_scatter():
        pltpu.make_async_copy(r0, acc.at[dstbA.at[0]], ssem).wait()

    def _issue_idx(base_chunk, swb, dstb):
        pltpu.async_copy(sw2d.at[pl.ds(base_chunk, _GB)], swb, isem)
        pltpu.async_copy(dst2d.at[pl.ds(base_chunk, _GB)], dstb, isem)

    def _wait_idx(base_chunk, swb, dstb):
        pltpu.make_async_copy(sw2d.at[pl.ds(base_chunk, _GB)], swb, isem).wait()
        pltpu.make_async_copy(dst2d.at[pl.ds(base_chunk, _GB)], dstb, isem).wait()

    p0 = s * _PAIR_CHUNKS
    _issue_idx(p0, swbA, dstbA)
    _wait_idx(p0, swbA, dstbA)
    _shift_src(swbA)
    for x in range(3):
        _issue_gather(x, swbA, x)

    def _pair(i, carry):
        pt = s + _NS * i
        c0 = pt * _PAIR_CHUNKS
        more = i < ni - 1
        cnext = c0 + _NS * _PAIR_CHUNKS

        def loc(x):
            return (swbA, dstbA, x) if x < _GB else (swbB, dstbB, x - _GB)

        for x in (0, 1):
            swb, dstb, row = loc(x)
            _wait_gather(x, swb, row)
            _issue_gather(x + 3, swbA, x + 3)
            _scale_chunk(rbuf[x % _GB], swb, row)
            _issue_scatter(x, dstb, row)

        _wait_gather(2, swbA, 2)
        _issue_idx(c0 + _GB, swbB, dstbB)
        _scale_chunk(r2, swbA, 2)
        _issue_scatter(2, dstbA, 2)

        _wait_gather(3, swbA, 3)
        _wait_idx(c0 + _GB, swbB, dstbB)
        _shift_src(swbB)
        _wait_scatter()
        _issue_gather(5, swbB, 0)
        _scale_chunk(r3, swbA, 3)
        _issue_scatter(3, dstbA, 3)

        for x in (4, 5, 6, 7):
            swb, dstb, row = loc(x)
            _wait_gather(x, swb, row)
            _wait_scatter()
            nswb, _, nrow = loc(x + 2)
            _issue_gather(x + 2, nswb, nrow)
            if x == 7:
                @pl.when(more)
                def _():
                    _issue_idx(cnext, swbA, dstbA)
            _scale_chunk(rbuf[x % _GB], swb, row)
            _issue_scatter(x, dstb, row)

        _wait_gather(8, swbB, 3)
        _wait_scatter()

        @pl.when(more)
        def _():
            _wait_idx(cnext, swbA, dstbA)
            _shift_src(swbA)
            _issue_gather(0, swbA, 0)
        _scale_chunk(r3, swbB, 3)
        _issue_scatter(8, dstbB, 3)

        _wait_gather(9, swbB, 4)
        _wait_scatter()

        @pl.when(more)
        def _():
            _issue_gather(1, swbA, 1)
        _scale_chunk(r4, swbB, 4)
        _issue_scatter(9, dstbB, 4)

        _wait_scatter()

        @pl.when(more)
        def _():
            _issue_gather(2, swbA, 2)
        _wait_scatter()
        _wait_scatter()
        return carry
    lax.fori_loop(0, ni, _pair, 0)

    plsc.subcore_barrier()

    no = (_FULL_OUT_CHUNKS - s + _NS - 1) // _NS
    def _out_chunk(i, carry):
        rr = (s + _NS * i) * _CHUNK
        pltpu.async_copy(acc.at[pl.ds(rr, _CHUNK)],
                         out_hbm.at[pl.ds(xbase + rr, _CHUNK)], gsem)
        return carry
    lax.fori_loop(0, no, _out_chunk, 0)

    @pl.when(s == _FULL_OUT_CHUNKS % _NS)
    def _():
        rr = _FULL_OUT_CHUNKS * _CHUNK
        pltpu.async_copy(acc.at[pl.ds(rr, _OUT_REM)],
                         out_hbm.at[pl.ds(xbase + rr, _OUT_REM)], gsem)

    def _out_drain(i, carry):
        pltpu.make_async_copy(acc.at[pl.ds(0, _CHUNK)],
                              out_hbm.at[pl.ds(0, _CHUNK)], gsem).wait()
        return carry
    lax.fori_loop(0, no, _out_drain, 0)

    @pl.when(s == _FULL_OUT_CHUNKS % _NS)
    def _():
        pltpu.make_async_copy(acc.at[pl.ds(0, _OUT_REM)],
                              out_hbm.at[pl.ds(0, _OUT_REM)], gsem).wait()


_spmm = functools.partial(
    pl.kernel,
    out_type=jax.ShapeDtypeStruct((2 * _N, _DH), jnp.float32),
    mesh=plsc.VectorSubcoreMesh(core_axis_name="c", subcore_axis_name="s"),
    compiler_params=pltpu.CompilerParams(use_tc_tiling_on_sc=False),
    scratch_types=[
        pltpu.VMEM((_GB, 2 * _CHUNK), jnp.int32),
        pltpu.VMEM((_GB, _CHUNK), jnp.int32),
        pltpu.VMEM((_GB, 2 * _CHUNK), jnp.int32),
        pltpu.VMEM((_GB, _CHUNK), jnp.int32),
        pltpu.VMEM((_CHUNK, _DH), jnp.float32),
        pltpu.VMEM((_CHUNK, _DH), jnp.float32),
        pltpu.VMEM((_CHUNK, _DH), jnp.float32),
        pltpu.VMEM((_CHUNK, _DH), jnp.float32),
        pltpu.VMEM((_CHUNK, _DH), jnp.float32),
        pltpu.VMEM_SHARED((_ACC_ROWS, _DH), jnp.float32),
        pltpu.SemaphoreType.DMA,
        pltpu.SemaphoreType.DMA,
        pltpu.SemaphoreType.DMA,
    ],
)(_spmm_body)


def _combine_body(x0_ref, lo2, hi2, lo3, hi3, lo4, hi4, o_ref):
    lo = lo2[...] + lo3[...] + lo4[...]
    hi = hi2[...] + hi3[...] + hi4[...]
    o_ref[...] = 0.25 * (x0_ref[...] + jnp.concatenate([lo, hi], axis=1))


_LO_SPEC = pl.BlockSpec((1000, _DH), lambda i: (i, 0))
_HI_SPEC = pl.BlockSpec((1000, _DH), lambda i: (i + _N // 1000, 0))

_combine = pl.pallas_call(
    _combine_body,
    out_shape=jax.ShapeDtypeStruct((_N, _D), jnp.float32),
    grid=(50,),
    in_specs=[pl.BlockSpec((1000, _D), lambda i: (i, 0)),
              _LO_SPEC, _HI_SPEC, _LO_SPEC, _HI_SPEC, _LO_SPEC, _HI_SPEC],
    out_specs=pl.BlockSpec((1000, _D), lambda i: (i, 0)),
)


def kernel(user_emb, item_emb, edge_index, edge_weight):
    x0 = jnp.concatenate([user_emb, item_emb], axis=0)
    src2d = edge_index[0].reshape(_NCHUNKS, _CHUNK)
    dst2d = edge_index[1].reshape(_NCHUNKS, _CHUNK)
    w2d = jax.lax.bitcast_convert_type(
        edge_weight.reshape(_NCHUNKS, _CHUNK), jnp.int32)
    sw2d = jnp.concatenate([src2d, w2d], axis=1)
    x0st = jnp.concatenate([x0[:, :_DH], x0[:, _DH:]], axis=0)
    x1st = _spmm(x0st, sw2d, dst2d)
    x2st = _spmm(x1st, sw2d, dst2d)
    x3st = _spmm(x2st, sw2d, dst2d)
    x4st = _spmm(x3st, sw2d, dst2d)
    light = _combine(x0, x2st, x2st, x3st, x3st, x4st, x4st)
    return light[:_NUM_USERS], light[_NUM_USERS:]
